# Initial kernel scaffold; baseline (speedup 1.0000x reference)
import jax
import jax.numpy as jnp
from jax.experimental import pallas as pl


def kernel(x, wavelength_shift):
    raise NotImplementedError("write your pallas kernel here")


# speedup vs baseline: 32.2152x; 32.2152x over previous
import jax
import jax.numpy as jnp
from jax.experimental import pallas as pl

_MAX_SHIFT_BANDS = 2.0


def _smile_kernel(x_ref, shift_ref, out_ref):
    x = x_ref[0]
    shift = shift_ref[0]
    Bh = x.shape[0]

    ci = jax.lax.broadcasted_iota(jnp.int32, shift.shape, 0)
    c = ci.astype(jnp.float32)
    s = jnp.clip(shift, -_MAX_SHIFT_BANDS, _MAX_SHIFT_BANDS)
    shifted = jnp.clip(c + s, 0.0, Bh - 1.0)
    pix = shifted * (float(Bh) / float(Bh - 1)) - 0.5
    pix = jnp.clip(pix, 0.0, Bh - 1.0)
    i0f = jnp.floor(pix)
    frac = pix - i0f
    i0 = i0f.astype(jnp.int32)
    i1 = jnp.minimum(i0 + 1, Bh - 1)

    d0 = i0 - ci
    d1 = i1 - ci

    acc = jnp.zeros_like(x)
    for d in range(-3, 4):
        wd = (jnp.where(d0 == d, 1.0 - frac, 0.0)
              + jnp.where(d1 == d, frac, 0.0))
        xs = x if d == 0 else jnp.roll(x, -d, axis=0)
        acc = acc + wd[:, None, :] * xs
    out_ref[0] = acc


def kernel(x, wavelength_shift):
    B, Bh, H, W = x.shape
    HBLK = 32
    grid = (B, H // HBLK)
    return pl.pallas_call(
        _smile_kernel,
        grid=grid,
        in_specs=[
            pl.BlockSpec((1, Bh, HBLK, W), lambda b, h: (b, 0, h, 0)),
            pl.BlockSpec((1, Bh, W), lambda b, h: (b, 0, 0)),
        ],
        out_specs=pl.BlockSpec((1, Bh, HBLK, W), lambda b, h: (b, 0, h, 0)),
        out_shape=jax.ShapeDtypeStruct((B, Bh, H, W), x.dtype),
    )(x, wavelength_shift)


# speedup vs baseline: 36.4520x; 1.1315x over previous
import jax
import jax.numpy as jnp
from jax.experimental import pallas as pl

_MAX_SHIFT_BANDS = 2.0


def _smile_kernel(x_ref, shift_ref, out_ref):
    x = x_ref[0]
    shift = shift_ref[0]
    Bh = x.shape[0]

    ci = jax.lax.broadcasted_iota(jnp.int32, shift.shape, 0)
    c = ci.astype(jnp.float32)
    s = jnp.clip(shift, -_MAX_SHIFT_BANDS, _MAX_SHIFT_BANDS)
    shifted = jnp.clip(c + s, 0.0, Bh - 1.0)
    pix = shifted * (float(Bh) / float(Bh - 1)) - 0.5
    pix = jnp.clip(pix, 0.0, Bh - 1.0)
    i0f = jnp.floor(pix)
    frac = pix - i0f
    i0 = i0f.astype(jnp.int32)
    i1 = jnp.minimum(i0 + 1, Bh - 1)

    d0 = i0 - ci
    d1 = i1 - ci

    acc = jnp.zeros_like(x)
    for d in range(-3, 4):
        wd = (jnp.where(d0 == d, 1.0 - frac, 0.0)
              + jnp.where(d1 == d, frac, 0.0))
        xs = x if d == 0 else jnp.roll(x, -d, axis=0)
        acc = acc + wd[:, None, :] * xs
    out_ref[0] = acc


def kernel(x, wavelength_shift):
    B, Bh, H, W = x.shape
    HBLK = 64
    grid = (B, H // HBLK)
    return pl.pallas_call(
        _smile_kernel,
        grid=grid,
        in_specs=[
            pl.BlockSpec((1, Bh, HBLK, W), lambda b, h: (b, 0, h, 0)),
            pl.BlockSpec((1, Bh, W), lambda b, h: (b, 0, 0)),
        ],
        out_specs=pl.BlockSpec((1, Bh, HBLK, W), lambda b, h: (b, 0, h, 0)),
        out_shape=jax.ShapeDtypeStruct((B, Bh, H, W), x.dtype),
    )(x, wavelength_shift)


# speedup vs baseline: 36.5624x; 1.0030x over previous
import jax
import jax.numpy as jnp
from jax.experimental import pallas as pl
from jax.experimental.pallas import tpu as pltpu

_MAX_SHIFT_BANDS = 2.0


def _smile_kernel(x_ref, shift_ref, out_ref):
    x = x_ref[0]
    shift = shift_ref[0]
    Bh = x.shape[0]

    ci = jax.lax.broadcasted_iota(jnp.int32, shift.shape, 0)
    c = ci.astype(jnp.float32)
    s = jnp.clip(shift, -_MAX_SHIFT_BANDS, _MAX_SHIFT_BANDS)
    shifted = jnp.clip(c + s, 0.0, Bh - 1.0)
    pix = shifted * (float(Bh) / float(Bh - 1)) - 0.5
    pix = jnp.clip(pix, 0.0, Bh - 1.0)
    i0f = jnp.floor(pix)
    frac = pix - i0f
    i0 = i0f.astype(jnp.int32)
    i1 = jnp.minimum(i0 + 1, Bh - 1)

    d0 = i0 - ci
    d1 = i1 - ci

    acc = jnp.zeros_like(x)
    for d in range(-3, 4):
        wd = (jnp.where(d0 == d, 1.0 - frac, 0.0)
              + jnp.where(d1 == d, frac, 0.0))
        xs = x if d == 0 else jnp.roll(x, -d, axis=0)
        acc = acc + wd[:, None, :] * xs
    out_ref[0] = acc


def kernel(x, wavelength_shift):
    B, Bh, H, W = x.shape
    HBLK = 64
    grid = (B, H // HBLK)
    return pl.pallas_call(
        _smile_kernel,
        grid=grid,
        in_specs=[
            pl.BlockSpec((1, Bh, HBLK, W), lambda b, h: (b, 0, h, 0)),
            pl.BlockSpec((1, Bh, W), lambda b, h: (b, 0, 0)),
        ],
        out_specs=pl.BlockSpec((1, Bh, HBLK, W), lambda b, h: (b, 0, h, 0)),
        out_shape=jax.ShapeDtypeStruct((B, Bh, H, W), x.dtype),
        compiler_params=pltpu.CompilerParams(
            dimension_semantics=("parallel", "parallel")),
    )(x, wavelength_shift)
